# baseline (speedup 1.0000x reference)
import functools

import jax
import jax.numpy as jnp
from jax.experimental import pallas as pl
from jax.experimental.pallas import tpu as pltpu
from jax.experimental.pallas import tpu_sc as plsc

B, T, D = 32, 576, 256
N_SUB, K, SUB = 8, 256, 32
HIDDEN = 1024
CLASSES = 1000
BT = B * T
TM = 2048
HALF = 1024
NK = N_SUB * K

N_SUBCORES = 32
CHUNK = BT // N_SUBCORES
GROUPS = CHUNK // 16


def _enc_body(z_ref, cbtbd_ref, csq_ref, out_ref):
    iota = jax.lax.broadcasted_iota(jnp.int32, (HALF, K), 1)
    for h0 in range(0, TM, HALF):
        rows = pl.ds(h0, HALF)
        zt = z_ref[rows, :]
        codes = []
        for n in range(N_SUB):
            dist = jnp.dot(zt, cbtbd_ref[:, n * K:(n + 1) * K],
                           preferred_element_type=jnp.float32) \
                + csq_ref[:, n * K:(n + 1) * K]
            m = jnp.min(dist, axis=1, keepdims=True)
            code = jnp.min(jnp.where(dist == m, iota, K), axis=1,
                           keepdims=True)
            codes.append(code)
        out_ref[rows, :] = jnp.concatenate(codes, axis=1)


def _sc_decode_body(codes_hbm, cb_hbm, out_hbm,
                    codes_v, cb_v, recon_v, sem_in, sem_cb, sem_out):
    c = jax.lax.axis_index("c")
    s = jax.lax.axis_index("s")
    sub_id = c * 16 + s
    start = sub_id * CHUNK

    cb_cp = pltpu.make_async_copy(cb_hbm, cb_v, sem_cb)
    cb_cp.start()
    codes_cp = pltpu.make_async_copy(
        codes_hbm.at[pl.ds(start * N_SUB, CHUNK * N_SUB)], codes_v, sem_in)
    codes_cp.start()
    cb_cp.wait()
    codes_cp.wait()

    lane = jax.lax.iota(jnp.int32, 16)

    def group(i, _):
        t0 = i * 16
        tids = t0 + lane
        for n in range(N_SUB):
            g = plsc.load_gather(codes_v, [tids * N_SUB + n])
            base = g * SUB + (n * K * SUB)
            for j in range(SUB):
                vals = plsc.load_gather(cb_v, [base + j])
                plsc.store_scatter(recon_v, [lane * D + (n * SUB + j)],
                                   vals)
        out_cp = pltpu.make_async_copy(
            recon_v, out_hbm.at[pl.ds((start + t0) * D, 16 * D)], sem_out)
        out_cp.start()
        out_cp.wait()
        return _

    jax.lax.fori_loop(0, GROUPS, group, None)


def _mlp_body(recon_ref, w1_ref, b1_ref, w2_ref, b2_ref, out_ref):
    for h0 in range(0, TM, HALF):
        rows = pl.ds(h0, HALF)
        q = recon_ref[rows, :].astype(jnp.bfloat16)
        h = jnp.maximum(jnp.dot(q, w1_ref[...],
                                preferred_element_type=jnp.float32)
                        + b1_ref[...], 0.0)
        out_ref[rows, :] = jnp.dot(h.astype(jnp.bfloat16), w2_ref[...],
                                   preferred_element_type=jnp.float32) \
            + b2_ref[...]


@jax.jit
def kernel(z, codebook, W1, b1, W2, b2):
    b, t, d = z.shape
    n_sub, k, sub = codebook.shape
    bt = b * t
    z2 = z.reshape(bt, d)
    cbT = codebook.transpose(0, 2, 1)
    cbtbd = jnp.concatenate(
        [jnp.pad(-2.0 * cbT[n], ((0, 0), (n * k, (n_sub - 1 - n) * k)))
         for n in range(n_sub)], axis=0)
    csq = jnp.sum(codebook * codebook, axis=-1).reshape(1, n_sub * k)
    cbflat = codebook.reshape(n_sub * k * sub)

    codes = pl.pallas_call(
        _enc_body,
        grid=(bt // TM,),
        in_specs=[
            pl.BlockSpec((TM, d), lambda i: (i, 0)),
            pl.BlockSpec((d, NK), lambda i: (0, 0)),
            pl.BlockSpec((1, NK), lambda i: (0, 0)),
        ],
        out_specs=pl.BlockSpec((TM, N_SUB), lambda i: (i, 0)),
        out_shape=jax.ShapeDtypeStruct((bt, N_SUB), jnp.int32),
        compiler_params=pltpu.CompilerParams(
            dimension_semantics=("arbitrary",),
        ),
    )(z2, cbtbd, csq)

    sc_decode = pl.kernel(
        _sc_decode_body,
        out_type=jax.ShapeDtypeStruct((bt * d,), jnp.float32),
        mesh=plsc.VectorSubcoreMesh(core_axis_name="c",
                                    subcore_axis_name="s"),
        compiler_params=pltpu.CompilerParams(needs_layout_passes=False),
        scratch_types=[
            pltpu.VMEM((CHUNK * N_SUB,), jnp.int32),
            pltpu.VMEM((NK * SUB,), jnp.float32),
            pltpu.VMEM((16 * D,), jnp.float32),
            pltpu.SemaphoreType.DMA,
            pltpu.SemaphoreType.DMA,
            pltpu.SemaphoreType.DMA,
        ],
    )
    recon = sc_decode(codes.reshape(bt * N_SUB), cbflat).reshape(bt, d)

    W1b = W1.astype(jnp.bfloat16)
    W2b = W2.astype(jnp.bfloat16)
    out = pl.pallas_call(
        _mlp_body,
        grid=(bt // TM,),
        in_specs=[
            pl.BlockSpec((TM, d), lambda i: (i, 0)),
            pl.BlockSpec((d, HIDDEN), lambda i: (0, 0)),
            pl.BlockSpec((1, HIDDEN), lambda i: (0, 0)),
            pl.BlockSpec((HIDDEN, CLASSES), lambda i: (0, 0)),
            pl.BlockSpec((1, CLASSES), lambda i: (0, 0)),
        ],
        out_specs=pl.BlockSpec((TM, CLASSES), lambda i: (i, 0)),
        out_shape=jax.ShapeDtypeStruct((bt, CLASSES), jnp.float32),
        compiler_params=pltpu.CompilerParams(
            dimension_semantics=("arbitrary",),
        ),
    )(recon, W1b, b1.reshape(1, HIDDEN), W2b, b2.reshape(1, CLASSES))
    return out.reshape(b, t, CLASSES)


# speedup vs baseline: 2.5133x; 2.5133x over previous
import jax
import jax.numpy as jnp
from jax.experimental import pallas as pl
from jax.experimental.pallas import tpu as pltpu

B, T, D = 32, 576, 256
N_SUB, K, SUB = 8, 256, 32
HIDDEN = 1024
CLASSES = 1000
TM = 3072
HALF = 1024
NK = N_SUB * K


def _body(z_ref, cbtbd_ref, csq_ref, cbs_ref, w1_ref, b1_ref, w2_ref, b2_ref,
          out_ref, onehot_scr):
    iota = jax.lax.broadcasted_iota(jnp.int32, (HALF, K), 1)
    for h0 in range(0, TM, HALF):
        rows = pl.ds(h0, HALF)
        zt = z_ref[rows, :]
        for n in range(N_SUB):
            dist = jnp.dot(zt, cbtbd_ref[:, n * K:(n + 1) * K],
                           preferred_element_type=jnp.float32) \
                + csq_ref[:, n * K:(n + 1) * K]
            m = jnp.min(dist, axis=1, keepdims=True)
            code = jnp.min(jnp.where(dist == m, iota, K), axis=1,
                           keepdims=True)
            onehot_scr[rows, n * K:(n + 1) * K] = (
                iota == code).astype(jnp.bfloat16)
        q = jnp.dot(onehot_scr[rows, :], cbs_ref[...],
                    preferred_element_type=jnp.float32)
        h = jnp.maximum(jnp.dot(q.astype(jnp.bfloat16), w1_ref[...],
                                preferred_element_type=jnp.float32)
                        + b1_ref[...], 0.0)
        out_ref[rows, :] = jnp.dot(h.astype(jnp.bfloat16), w2_ref[...],
                                   preferred_element_type=jnp.float32) \
            + b2_ref[...]


@jax.jit
def kernel(z, codebook, W1, b1, W2, b2):
    b, t, d = z.shape
    n_sub, k, sub = codebook.shape
    bt = b * t
    z2 = z.reshape(bt, d)
    cbT = codebook.transpose(0, 2, 1)
    cbtbd = jnp.concatenate(
        [jnp.pad(-2.0 * cbT[n], ((0, 0), (n * k, (n_sub - 1 - n) * k)))
         for n in range(n_sub)], axis=0)
    csq = jnp.sum(codebook * codebook, axis=-1).reshape(1, n_sub * k)
    cbs = jnp.concatenate(
        [jnp.pad(codebook[n], ((0, 0), (n * sub, d - (n + 1) * sub)))
         for n in range(n_sub)], axis=0).astype(jnp.bfloat16)
    W1b = W1.astype(jnp.bfloat16)
    W2b = W2.astype(jnp.bfloat16)
    b2r = b2.reshape(1, CLASSES)
    b1r = b1.reshape(1, HIDDEN)

    grid = (bt // TM,)
    out = pl.pallas_call(
        _body,
        grid=grid,
        in_specs=[
            pl.BlockSpec((TM, d), lambda i: (i, 0)),
            pl.BlockSpec((d, NK), lambda i: (0, 0)),
            pl.BlockSpec((1, NK), lambda i: (0, 0)),
            pl.BlockSpec((NK, d), lambda i: (0, 0)),
            pl.BlockSpec((d, HIDDEN), lambda i: (0, 0)),
            pl.BlockSpec((1, HIDDEN), lambda i: (0, 0)),
            pl.BlockSpec((HIDDEN, CLASSES), lambda i: (0, 0)),
            pl.BlockSpec((1, CLASSES), lambda i: (0, 0)),
        ],
        out_specs=pl.BlockSpec((TM, CLASSES), lambda i: (i, 0)),
        out_shape=jax.ShapeDtypeStruct((bt, CLASSES), jnp.float32),
        scratch_shapes=[pltpu.VMEM((TM, NK), jnp.bfloat16)],
        compiler_params=pltpu.CompilerParams(
            dimension_semantics=("arbitrary",),
        ),
    )(z2, cbtbd, csq, cbs, W1b, b1r, W2b, b2r)
    return out.reshape(b, t, CLASSES)
